# Initial kernel scaffold; baseline (speedup 1.0000x reference)
import jax
import jax.numpy as jnp
from jax.experimental import pallas as pl


def kernel(x, time_step, batch, alphas_bar):
    raise NotImplementedError("write your pallas kernel here")


# speedup vs baseline: 7.7661x; 7.7661x over previous
import functools

import jax
import jax.numpy as jnp
from jax import lax
from jax.experimental import pallas as pl
from jax.experimental.pallas import tpu as pltpu
from jax.experimental.pallas import tpu_sc as plsc

NUM_CLASSES = 16
_LANES = 128
_ROWS_PER_BLOCK = 512


def _tc_body(x_ref, ab_ref, pert_ref, oh_ref):
    rb = pert_ref.shape[0]
    r0 = pl.program_id(0) * rb
    row = lax.broadcasted_iota(jnp.int32, (rb, _LANES), 0)
    lane = lax.broadcasted_iota(jnp.int32, (rb, _LANES), 1)
    lin = (r0 + row) * _LANES + lane

    ks0 = jnp.uint32(0)
    ks1 = jnp.uint32(42)
    ks2 = ks0 ^ ks1 ^ jnp.uint32(0x1BD11BDA)
    ks = (ks0, ks1, ks2)
    x0 = jnp.full((rb, _LANES), ks0, jnp.uint32)
    x1 = lin.astype(jnp.uint32) + ks1
    rot = ((13, 15, 26, 6), (17, 29, 16, 24))
    for g in range(5):
        for d in rot[g % 2]:
            x0 = x0 + x1
            x1 = (x1 << jnp.uint32(d)) | (x1 >> jnp.uint32(32 - d))
            x1 = x0 ^ x1
        x0 = x0 + ks[(g + 1) % 3]
        x1 = x1 + ks[(g + 2) % 3] + jnp.uint32(g + 1)
    bits = x0 ^ x1

    fb = (bits >> jnp.uint32(9)) | jnp.uint32(0x3F800000)
    f = lax.bitcast_convert_type(fb, jnp.float32)
    lo = jnp.float32(-0.99999994)
    span = jnp.float32(1.0) - lo
    u = jnp.maximum(lo, (f - jnp.float32(1.0)) * span + lo)

    w = -jnp.log1p(-u * u)
    w1 = w - jnp.float32(2.5)
    p1 = jnp.float32(2.81022636e-08)
    for c in (3.43273939e-07, -3.5233877e-06, -4.39150654e-06, 2.1858087e-04,
              -1.25372503e-03, -4.17768164e-03, 2.46640727e-01, 1.50140941e+00):
        p1 = jnp.float32(c) + p1 * w1
    w2 = jnp.sqrt(w) - jnp.float32(3.0)
    p2 = jnp.float32(-2.00214257e-04)
    for c in (1.00950558e-04, 1.34934322e-03, -3.67342844e-03, 5.73950773e-03,
              -7.6224613e-03, 9.43887047e-03, 1.00167406e+00, 2.83297682e+00):
        p2 = jnp.float32(c) + p2 * w2
    z = jnp.where(w < jnp.float32(5.0), p1, p2) * u * jnp.float32(1.4142135623730951)

    kio = lax.broadcasted_iota(jnp.int32, (8, _LANES), 0)
    cio = lax.broadcasted_iota(jnp.int32, (8, _LANES), 1)
    sel = ((cio >> 4) == kio).astype(jnp.float32)
    xrep = jnp.dot(x_ref[...].astype(jnp.float32), sel,
                   preferred_element_type=jnp.float32)
    abrep = jnp.dot(ab_ref[...], sel, preferred_element_type=jnp.float32)

    oh = ((lane & 15) == xrep.astype(jnp.int32)).astype(jnp.float32)
    pert_ref[...] = jnp.sqrt(abrep) * oh + jnp.sqrt(jnp.float32(1.0) - abrep) * z
    oh_ref[...] = oh


def _tc_call(x2d, ab2d):
    m = x2d.shape[0]
    rb = _ROWS_PER_BLOCK
    grid = (m // rb,)
    return pl.pallas_call(
        _tc_body,
        grid=grid,
        in_specs=[
            pl.BlockSpec((rb, 8), lambda i: (i, 0)),
            pl.BlockSpec((rb, 8), lambda i: (i, 0)),
        ],
        out_specs=[
            pl.BlockSpec((rb, _LANES), lambda i: (i, 0)),
            pl.BlockSpec((rb, _LANES), lambda i: (i, 0)),
        ],
        out_shape=[
            jax.ShapeDtypeStruct((m, _LANES), jnp.float32),
            jax.ShapeDtypeStruct((m, _LANES), jnp.float32),
        ],
        compiler_params=pltpu.CompilerParams(
            dimension_semantics=("arbitrary",),
        ),
    )(x2d, ab2d)


def _sc_double_gather(time_step, batch, alphas_pad):
    n = batch.shape[0]
    info = plsc.get_sparse_core_info()
    nc, ns = info.num_cores, info.num_subcores
    nw = nc * ns
    chunk = n // nw
    b = time_step.shape[0]
    tpad = alphas_pad.shape[0]
    mesh = plsc.VectorSubcoreMesh(core_axis_name="c", subcore_axis_name="s")

    @functools.partial(
        pl.kernel,
        mesh=mesh,
        out_type=jax.ShapeDtypeStruct((n,), jnp.float32),
        scratch_types=[
            pltpu.VMEM((chunk,), jnp.int32),
            pltpu.VMEM((chunk,), jnp.float32),
            pltpu.VMEM((b,), jnp.int32),
            pltpu.VMEM((tpad,), jnp.float32),
        ],
        compiler_params=pltpu.CompilerParams(needs_layout_passes=False),
    )
    def k(ts_hbm, batch_hbm, al_hbm, out_hbm, idx_v, ab_v, ts_v, al_v):
        wid = lax.axis_index("s") * nc + lax.axis_index("c")
        base = wid * chunk
        pltpu.sync_copy(ts_hbm, ts_v)
        pltpu.sync_copy(al_hbm, al_v)
        pltpu.sync_copy(batch_hbm.at[pl.ds(base, chunk)], idx_v)

        def body(i, carry):
            o = i * 64
            for j in range(4):
                idx = idx_v[pl.ds(o + j * 16, 16)]
                t = plsc.load_gather(ts_v, [idx])
                a = plsc.load_gather(al_v, [t])
                ab_v[pl.ds(o + j * 16, 16)] = a
            return carry

        lax.fori_loop(0, chunk // 64, body, 0)
        pltpu.sync_copy(ab_v, out_hbm.at[pl.ds(base, chunk)])

    return k(time_step, batch, alphas_pad)


def kernel(x, time_step, batch, alphas_bar):
    n = x.shape[0]
    t = alphas_bar.shape[0]
    x = x.astype(jnp.int32)
    time_step = time_step.astype(jnp.int32)
    batch = batch.astype(jnp.int32)
    alphas_bar = alphas_bar.astype(jnp.float32)
    alphas_pad = jnp.pad(alphas_bar, (0, (-t) % 16))
    ab_row = _sc_double_gather(time_step, batch, alphas_pad)
    pert_f, oh_f = _tc_call(x.reshape(n // 8, 8), ab_row.reshape(n // 8, 8))
    return pert_f.reshape(n, NUM_CLASSES), oh_f.reshape(n, NUM_CLASSES)


# speedup vs baseline: 22.4757x; 2.8941x over previous
import functools

import jax
import jax.numpy as jnp
from jax import lax
from jax.experimental import pallas as pl
from jax.experimental.pallas import tpu as pltpu
from jax.experimental.pallas import tpu_sc as plsc

NUM_CLASSES = 16
_LANES = 128
_TILES_PER_BLOCK = 32


def _noise(lin):
    ks0 = jnp.uint32(0)
    ks1 = jnp.uint32(42)
    ks2 = ks0 ^ ks1 ^ jnp.uint32(0x1BD11BDA)
    ks = (ks0, ks1, ks2)
    x0 = jnp.zeros_like(lin, jnp.uint32) + ks0
    x1 = lin.astype(jnp.uint32) + ks1
    rot = ((13, 15, 26, 6), (17, 29, 16, 24))
    for g in range(5):
        for d in rot[g % 2]:
            x0 = x0 + x1
            x1 = (x1 << jnp.uint32(d)) | (x1 >> jnp.uint32(32 - d))
            x1 = x0 ^ x1
        x0 = x0 + ks[(g + 1) % 3]
        x1 = x1 + ks[(g + 2) % 3] + jnp.uint32(g + 1)
    bits = x0 ^ x1

    fb = (bits >> jnp.uint32(9)) | jnp.uint32(0x3F800000)
    f = lax.bitcast_convert_type(fb, jnp.float32)
    lo = jnp.float32(-0.99999994)
    span = jnp.float32(1.0) - lo
    u = jnp.maximum(lo, (f - jnp.float32(1.0)) * span + lo)

    w = -jnp.log1p(-u * u)
    w1 = w - jnp.float32(2.5)
    p1 = jnp.float32(2.81022636e-08)
    for c in (3.43273939e-07, -3.5233877e-06, -4.39150654e-06, 2.1858087e-04,
              -1.25372503e-03, -4.17768164e-03, 2.46640727e-01, 1.50140941e+00):
        p1 = jnp.float32(c) + p1 * w1
    w2 = jnp.sqrt(w) - jnp.float32(3.0)
    p2 = jnp.float32(-2.00214257e-04)
    for c in (1.00950558e-04, 1.34934322e-03, -3.67342844e-03, 5.73950773e-03,
              -7.6224613e-03, 9.43887047e-03, 1.00167406e+00, 2.83297682e+00):
        p2 = jnp.float32(c) + p2 * w2
    return jnp.where(w < jnp.float32(5.0), p1, p2) * u * jnp.float32(1.4142135623730951)


def _tc_body(x_ref, ab_ref, pert_ref, oh_ref):
    r_tiles = x_ref.shape[0]
    cn = r_tiles * _LANES
    a0 = pl.program_id(0) * cn

    icls = lax.broadcasted_iota(jnp.int32, (NUM_CLASSES, cn), 0)
    iatom = lax.broadcasted_iota(jnp.int32, (NUM_CLASSES, cn), 1)
    z = _noise((a0 + iatom) * NUM_CLASSES + icls)

    sub8 = lax.broadcasted_iota(jnp.int32, (8, _LANES), 0)
    for r in range(r_tiles):
        x8 = jnp.broadcast_to(x_ref[r:r + 1, :], (8, _LANES))
        ab8 = jnp.broadcast_to(ab_ref[r:r + 1, :], (8, _LANES))
        sa = jnp.sqrt(ab8)
        sb = jnp.sqrt(jnp.float32(1.0) - ab8)
        for ch in range(2):
            oh = (x8 == sub8 + 8 * ch).astype(jnp.float32)
            zs = z[ch * 8:(ch + 1) * 8, r * _LANES:(r + 1) * _LANES]
            pert_ref[ch * 8:(ch + 1) * 8, r * _LANES:(r + 1) * _LANES] = (
                sa * oh + sb * zs)
            oh_ref[ch * 8:(ch + 1) * 8, r * _LANES:(r + 1) * _LANES] = oh


def _tc_call(x3, ab3):
    rows = x3.shape[0]
    n = rows * _LANES
    rb = _TILES_PER_BLOCK
    cn = rb * _LANES
    return pl.pallas_call(
        _tc_body,
        grid=(rows // rb,),
        in_specs=[
            pl.BlockSpec((rb, _LANES), lambda i: (i, 0)),
            pl.BlockSpec((rb, _LANES), lambda i: (i, 0)),
        ],
        out_specs=[
            pl.BlockSpec((NUM_CLASSES, cn), lambda i: (0, i)),
            pl.BlockSpec((NUM_CLASSES, cn), lambda i: (0, i)),
        ],
        out_shape=[
            jax.ShapeDtypeStruct((NUM_CLASSES, n), jnp.float32),
            jax.ShapeDtypeStruct((NUM_CLASSES, n), jnp.float32),
        ],
        compiler_params=pltpu.CompilerParams(
            dimension_semantics=("arbitrary",),
        ),
    )(x3, ab3)


def _sc_double_gather(time_step, batch, alphas_pad):
    n = batch.shape[0]
    info = plsc.get_sparse_core_info()
    nc, ns = info.num_cores, info.num_subcores
    nw = nc * ns
    chunk = n // nw
    b = time_step.shape[0]
    tpad = alphas_pad.shape[0]
    mesh = plsc.VectorSubcoreMesh(core_axis_name="c", subcore_axis_name="s")

    @functools.partial(
        pl.kernel,
        mesh=mesh,
        out_type=jax.ShapeDtypeStruct((n,), jnp.float32),
        scratch_types=[
            pltpu.VMEM((chunk,), jnp.int32),
            pltpu.VMEM((chunk,), jnp.float32),
            pltpu.VMEM((b,), jnp.int32),
            pltpu.VMEM((tpad,), jnp.float32),
        ],
        compiler_params=pltpu.CompilerParams(needs_layout_passes=False),
    )
    def k(ts_hbm, batch_hbm, al_hbm, out_hbm, idx_v, ab_v, ts_v, al_v):
        wid = lax.axis_index("s") * nc + lax.axis_index("c")
        base = wid * chunk
        pltpu.sync_copy(ts_hbm, ts_v)
        pltpu.sync_copy(al_hbm, al_v)
        pltpu.sync_copy(batch_hbm.at[pl.ds(base, chunk)], idx_v)

        def body(i, carry):
            o = i * 64
            for j in range(4):
                idx = idx_v[pl.ds(o + j * 16, 16)]
                t = plsc.load_gather(ts_v, [idx])
                a = plsc.load_gather(al_v, [t])
                ab_v[pl.ds(o + j * 16, 16)] = a
            return carry

        lax.fori_loop(0, chunk // 64, body, 0)
        pltpu.sync_copy(ab_v, out_hbm.at[pl.ds(base, chunk)])

    return k(time_step, batch, alphas_pad)


def kernel(x, time_step, batch, alphas_bar):
    n = x.shape[0]
    t = alphas_bar.shape[0]
    x = x.astype(jnp.int32)
    time_step = time_step.astype(jnp.int32)
    batch = batch.astype(jnp.int32)
    alphas_bar = alphas_bar.astype(jnp.float32)
    alphas_pad = jnp.pad(alphas_bar, (0, (-t) % 16))
    ab_row = _sc_double_gather(time_step, batch, alphas_pad)
    pert_t, oh_t = _tc_call(x.reshape(n // _LANES, _LANES),
                            ab_row.reshape(n // _LANES, _LANES))
    return pert_t.T, oh_t.T
